# baseline (speedup 1.0000x reference)
import functools

import jax
import jax.numpy as jnp
from jax import lax
from jax.experimental import pallas as pl
from jax.experimental.pallas import tpu as pltpu


def _dn_kernel(x_ref, w_ref, b_ref, tw_ref, inv_ref, o_ref, *, H, W):
    x = x_ref[0]
    C = x.shape[0]

    x16 = x.astype(jnp.bfloat16)
    zh = jnp.zeros((C, 1, W), jnp.bfloat16)
    v = (x16 + jnp.concatenate([zh, x16[:, :-1, :]], axis=1)
             + jnp.concatenate([x16[:, 1:, :], zh], axis=1)) * inv_ref[...]

    v2 = v.reshape(C, H * W)
    t2 = lax.dot_general(w_ref[...], v2, (((1,), (0,)), ((), ())),
                         preferred_element_type=jnp.float32)
    t3 = t2.astype(jnp.bfloat16).reshape(C, H, W)

    s = lax.dot_general(t3.reshape(C * H, W), tw_ref[...],
                        (((1,), (0,)), ((), ())),
                        preferred_element_type=jnp.float32).reshape(C, H, W)

    o_ref[0] = x + s + b_ref[...]


def kernel(x, conv_w, conv_b):
    B, C, H, W = x.shape
    f32 = jnp.float32

    i = jnp.arange(W)
    cw = jnp.where((i == 0) | (i == W - 1), 2.0, 3.0)
    tw = ((jnp.abs(i[:, None] - i[None, :]) <= 1) / cw[None, :]) \
        .astype(jnp.bfloat16)
    ch = jnp.where((jnp.arange(H) == 0) | (jnp.arange(H) == H - 1), 2.0, 3.0)
    inv = jnp.broadcast_to((1.0 / ch)[:, None], (H, W)) \
        .astype(jnp.bfloat16)[None]
    b3 = conv_b.reshape(C, 1, 1)
    w16 = conv_w.astype(jnp.bfloat16)

    return pl.pallas_call(
        functools.partial(_dn_kernel, H=H, W=W),
        grid=(B,),
        in_specs=[
            pl.BlockSpec((1, C, H, W), lambda b: (b, 0, 0, 0)),
            pl.BlockSpec((C, C), lambda b: (0, 0)),
            pl.BlockSpec((C, 1, 1), lambda b: (0, 0, 0)),
            pl.BlockSpec((W, W), lambda b: (0, 0)),
            pl.BlockSpec((1, H, W), lambda b: (0, 0, 0)),
        ],
        out_specs=pl.BlockSpec((1, C, H, W), lambda b: (b, 0, 0, 0)),
        out_shape=jax.ShapeDtypeStruct((B, C, H, W), x.dtype),
        compiler_params=pltpu.CompilerParams(
            dimension_semantics=("parallel",),
        ),
    )(x, w16, b3, tw, inv)


# speedup vs baseline: 1.0037x; 1.0037x over previous
import functools

import jax
import jax.numpy as jnp
from jax import lax
from jax.experimental import pallas as pl
from jax.experimental.pallas import tpu as pltpu


def _dn_kernel(x_ref, w_ref, b_ref, tw_ref, inv_ref, o_ref, *, H, W):
    x = x_ref[0]
    C = x.shape[0]

    x16 = x.astype(jnp.bfloat16)
    zh = jnp.zeros((C, 1, W), jnp.bfloat16)
    v = x16 + jnp.concatenate([zh, x16[:, :-1, :]], axis=1) \
            + jnp.concatenate([x16[:, 1:, :], zh], axis=1)

    v2 = v.reshape(C, H * W)
    t2 = lax.dot_general(w_ref[...], v2, (((1,), (0,)), ((), ())),
                         preferred_element_type=jnp.float32)
    t3 = t2.astype(jnp.bfloat16).reshape(C, H, W)

    s = lax.dot_general(t3.reshape(C * H, W), tw_ref[...],
                        (((1,), (0,)), ((), ())),
                        preferred_element_type=jnp.float32).reshape(C, H, W)

    o_ref[0] = x + s * inv_ref[...] + b_ref[...]


def kernel(x, conv_w, conv_b):
    B, C, H, W = x.shape
    f32 = jnp.float32

    i = jnp.arange(W)
    tw = (jnp.abs(i[:, None] - i[None, :]) <= 1).astype(jnp.bfloat16)
    ch = jnp.where((jnp.arange(H) == 0) | (jnp.arange(H) == H - 1), 2.0, 3.0)
    cw = jnp.where((i == 0) | (i == W - 1), 2.0, 3.0)
    inv = (1.0 / (ch[:, None] * cw[None, :])).astype(f32)[None]
    b3 = conv_b.reshape(C, 1, 1)
    w16 = conv_w.astype(jnp.bfloat16)

    return pl.pallas_call(
        functools.partial(_dn_kernel, H=H, W=W),
        grid=(B,),
        in_specs=[
            pl.BlockSpec((1, C, H, W), lambda b: (b, 0, 0, 0)),
            pl.BlockSpec((C, C), lambda b: (0, 0)),
            pl.BlockSpec((C, 1, 1), lambda b: (0, 0, 0)),
            pl.BlockSpec((W, W), lambda b: (0, 0)),
            pl.BlockSpec((1, H, W), lambda b: (0, 0, 0)),
        ],
        out_specs=pl.BlockSpec((1, C, H, W), lambda b: (b, 0, 0, 0)),
        out_shape=jax.ShapeDtypeStruct((B, C, H, W), x.dtype),
        compiler_params=pltpu.CompilerParams(
            dimension_semantics=("parallel",),
        ),
    )(x, w16, b3, tw, inv)


# speedup vs baseline: 1.0723x; 1.0683x over previous
import functools

import jax
import jax.numpy as jnp
from jax import lax
from jax.experimental import pallas as pl
from jax.experimental.pallas import tpu as pltpu


def _dn_kernel(x_ref, w_ref, b_ref, tw_ref, inv_ref, o_ref, *, H, W):
    G, C = x_ref.shape[0], x_ref.shape[1]
    R = G * C
    x = x_ref[...].reshape(R, H, W)

    x16 = x.astype(jnp.bfloat16)
    zh = jnp.zeros((R, 1, W), jnp.bfloat16)
    v = x16 + jnp.concatenate([zh, x16[:, :-1, :]], axis=1) \
            + jnp.concatenate([x16[:, 1:, :], zh], axis=1)

    v2 = v.reshape(R, H * W)
    t2 = lax.dot_general(w_ref[...], v2, (((1,), (0,)), ((), ())),
                         preferred_element_type=jnp.float32)
    t3 = t2.astype(jnp.bfloat16).reshape(R, H, W)

    s = lax.dot_general(t3.reshape(R * H, W), tw_ref[...],
                        (((1,), (0,)), ((), ())),
                        preferred_element_type=jnp.float32).reshape(R, H, W)

    o = x + s * inv_ref[...] + b_ref[...]
    o_ref[...] = o.reshape(G, C, H, W)


def kernel(x, conv_w, conv_b):
    B, C, H, W = x.shape
    f32 = jnp.float32
    G = 2

    i = jnp.arange(W)
    tw = (jnp.abs(i[:, None] - i[None, :]) <= 1).astype(jnp.bfloat16)
    ch = jnp.where((jnp.arange(H) == 0) | (jnp.arange(H) == H - 1), 2.0, 3.0)
    cw = jnp.where((i == 0) | (i == W - 1), 2.0, 3.0)
    inv = (1.0 / (ch[:, None] * cw[None, :])).astype(f32)[None]
    b3 = jnp.tile(conv_b, G).reshape(G * C, 1, 1)
    eye = jnp.eye(G, dtype=f32)
    wbd = (jnp.kron(eye, conv_w)).astype(jnp.bfloat16)

    return pl.pallas_call(
        functools.partial(_dn_kernel, H=H, W=W),
        grid=(B // G,),
        in_specs=[
            pl.BlockSpec((G, C, H, W), lambda b: (b, 0, 0, 0)),
            pl.BlockSpec((G * C, G * C), lambda b: (0, 0)),
            pl.BlockSpec((G * C, 1, 1), lambda b: (0, 0, 0)),
            pl.BlockSpec((W, W), lambda b: (0, 0)),
            pl.BlockSpec((1, H, W), lambda b: (0, 0, 0)),
        ],
        out_specs=pl.BlockSpec((G, C, H, W), lambda b: (b, 0, 0, 0)),
        out_shape=jax.ShapeDtypeStruct((B, C, H, W), x.dtype),
        compiler_params=pltpu.CompilerParams(
            dimension_semantics=("parallel",),
        ),
    )(x, wbd, b3, tw, inv)
